# Initial kernel scaffold; baseline (speedup 1.0000x reference)
import jax
import jax.numpy as jnp
from jax.experimental import pallas as pl


def kernel(idx, wte, wpe):
    raise NotImplementedError("write your pallas kernel here")


# speedup vs baseline: 1.0207x; 1.0207x over previous
import functools

import jax
import jax.numpy as jnp
from jax import lax
from jax.experimental import pallas as pl
from jax.experimental.pallas import tpu as pltpu
from jax.experimental.pallas import tpu_sc as plsc

_LANES = 16
_NUM_WORKERS = 32
_CHUNK = 64


@functools.lru_cache(maxsize=None)
def _build(B, T, D, nw):
    b_per_w = B // nw
    n_chunks = b_per_w // _CHUNK
    mesh = plsc.VectorSubcoreMesh(core_axis_name="c", subcore_axis_name="s")

    @functools.partial(
        pl.kernel,
        mesh=mesh,
        out_type=jax.ShapeDtypeStruct((B, D), jnp.float32),
        scratch_types=[
            pltpu.VMEM((_CHUNK,), jnp.int32),
            pltpu.VMEM((_CHUNK, D), jnp.float32),
            pltpu.VMEM((_CHUNK, D), jnp.float32),
            pltpu.SemaphoreType.DMA,
        ],
    )
    def sc_kernel(idx_hbm, wte_hbm, wpe_hbm, out_hbm, idx_v, rows_v, pos_v, sem):
        wid = lax.axis_index("s") * 2 + lax.axis_index("c")
        base = pl.multiple_of(wid * b_per_w, b_per_w)
        t0 = pl.multiple_of(lax.rem(base, T), b_per_w)
        for c in range(n_chunks):
            off = base + c * _CHUNK
            pltpu.sync_copy(idx_hbm.at[pl.ds(off, _CHUNK)], idx_v)
            gather = pltpu.async_copy(wte_hbm.at[idx_v], rows_v, sem)
            pltpu.sync_copy(wpe_hbm.at[pl.ds(t0 + c * _CHUNK, _CHUNK)], pos_v)
            gather.wait()

            def body(r, carry):
                for j in range(D // _LANES):
                    sl = pl.ds(j * _LANES, _LANES)
                    rows_v[r, sl] = rows_v[r, sl] + pos_v[r, sl]
                return carry

            lax.fori_loop(0, _CHUNK, body, 0)
            pltpu.sync_copy(rows_v, out_hbm.at[pl.ds(off, _CHUNK)])

    return sc_kernel


def kernel(idx, wte, wpe):
    b, t = idx.shape
    v, d = wte.shape
    B = b * t
    idx_flat = idx.reshape(B).astype(jnp.int32)
    out = _build(B, t, d, _NUM_WORKERS)(idx_flat, wte, wpe)
    return out.reshape(b, t, d)


# speedup vs baseline: 1.1686x; 1.1449x over previous
import functools

import jax
import jax.numpy as jnp
from jax import lax
from jax.experimental import pallas as pl
from jax.experimental.pallas import tpu as pltpu
from jax.experimental.pallas import tpu_sc as plsc

_LANES = 16
_NUM_WORKERS = 32
_CHUNK = 16
_DEPTH = 3


@functools.lru_cache(maxsize=None)
def _build(B, T, D, nw):
    b_per_w = B // nw
    n_items = b_per_w // _CHUNK
    mesh = plsc.VectorSubcoreMesh(core_axis_name="c", subcore_axis_name="s")

    @functools.partial(
        pl.kernel,
        mesh=mesh,
        out_type=jax.ShapeDtypeStruct((B, D), jnp.float32),
        scratch_types=[
            pltpu.VMEM((b_per_w,), jnp.int32),
            [pltpu.VMEM((_CHUNK, D), jnp.float32) for _ in range(_DEPTH)],
            [pltpu.VMEM((_CHUNK, D), jnp.float32) for _ in range(_DEPTH)],
            [pltpu.SemaphoreType.DMA for _ in range(_DEPTH)],
            [pltpu.SemaphoreType.DMA for _ in range(_DEPTH)],
            [pltpu.SemaphoreType.DMA for _ in range(_DEPTH)],
        ],
    )
    def sc_kernel(idx_hbm, wte_hbm, wpe_hbm, out_hbm, idx_v, outs, gaths,
                  fsems, gsems, ssems):
        wid = lax.axis_index("s") * 2 + lax.axis_index("c")
        base = pl.multiple_of(wid * b_per_w, b_per_w)
        t0 = pl.multiple_of(lax.rem(base, T), b_per_w)
        pltpu.sync_copy(idx_hbm.at[pl.ds(base, b_per_w)], idx_v)

        store_desc = [None] * _DEPTH

        def issue(k):
            s = k % _DEPTH
            if store_desc[s] is not None:
                store_desc[s].wait()
                store_desc[s] = None
            fill = pltpu.async_copy(
                wpe_hbm.at[pl.ds(t0 + k * _CHUNK, _CHUNK)], outs[s], fsems[s])
            gath = pltpu.async_copy(
                wte_hbm.at[idx_v.at[pl.ds(k * _CHUNK, _CHUNK)]], gaths[s],
                gsems[s])
            return fill, gath

        in_flight = [None] * _DEPTH
        for k in range(min(_DEPTH, n_items)):
            in_flight[k % _DEPTH] = issue(k)

        for j in range(n_items):
            s = j % _DEPTH
            fill, gath = in_flight[s]
            fill.wait()
            gath.wait()

            def body(r, carry):
                for q in range(D // _LANES):
                    sl = pl.ds(q * _LANES, _LANES)
                    plsc.addupdate(outs[s].at[r, sl], gaths[s][r, sl])
                return carry

            lax.fori_loop(0, _CHUNK, body, 0)
            store_desc[s] = pltpu.async_copy(
                outs[s], out_hbm.at[pl.ds(base + j * _CHUNK, _CHUNK)], ssems[s])
            if j + _DEPTH < n_items:
                in_flight[s] = issue(j + _DEPTH)

        for s in range(_DEPTH):
            if store_desc[s] is not None:
                store_desc[s].wait()

    return sc_kernel


def kernel(idx, wte, wpe):
    b, t = idx.shape
    v, d = wte.shape
    B = b * t
    idx_flat = idx.reshape(B).astype(jnp.int32)
    out = _build(B, t, d, _NUM_WORKERS)(idx_flat, wte, wpe)
    return out.reshape(b, t, d)


# speedup vs baseline: 1.2499x; 1.0696x over previous
import functools

import jax
import jax.numpy as jnp
from jax import lax
from jax.experimental import pallas as pl
from jax.experimental.pallas import tpu as pltpu
from jax.experimental.pallas import tpu_sc as plsc

_LANES = 16
_NUM_WORKERS = 32
_CHUNK = 16
_DEPTH = 3


@functools.lru_cache(maxsize=None)
def _build(B, T, D, n_batch, nw):
    b_per_w = B // nw
    t_span = b_per_w // n_batch
    n_sub = t_span // _CHUNK
    n_items = n_batch * n_sub
    mesh = plsc.VectorSubcoreMesh(core_axis_name="c", subcore_axis_name="s")

    @functools.partial(
        pl.kernel,
        mesh=mesh,
        out_type=jax.ShapeDtypeStruct((B, D), jnp.float32),
        scratch_types=[
            pltpu.VMEM((b_per_w,), jnp.int32),
            pltpu.VMEM((t_span, D), jnp.float32),
            [pltpu.VMEM((_CHUNK, D), jnp.float32) for _ in range(_DEPTH)],
            [pltpu.VMEM((_CHUNK, D), jnp.float32) for _ in range(_DEPTH)],
            [pltpu.SemaphoreType.DMA for _ in range(_DEPTH)],
            [pltpu.SemaphoreType.DMA for _ in range(_DEPTH)],
            pltpu.SemaphoreType.DMA,
        ],
    )
    def sc_kernel(idx_hbm, wte_hbm, wpe_hbm, out_hbm, idx_v, pos_v, gaths,
                  outs, gsems, ssems, isem):
        wid = lax.axis_index("s") * 2 + lax.axis_index("c")
        t0 = pl.multiple_of(wid * t_span, t_span)
        idx_cps = [
            pltpu.async_copy(idx_hbm.at[pl.ds(b * T + t0, t_span)],
                             idx_v.at[pl.ds(b * t_span, t_span)], isem)
            for b in range(n_batch)
        ]
        pltpu.sync_copy(wpe_hbm.at[pl.ds(t0, t_span)], pos_v)
        for cp in idx_cps:
            cp.wait()

        def issue_gather(k):
            s = k % _DEPTH
            b, tc = k // n_sub, k % n_sub
            return pltpu.async_copy(
                wte_hbm.at[idx_v.at[pl.ds(b * t_span + tc * _CHUNK, _CHUNK)]],
                gaths[s], gsems[s])

        in_flight = [None] * _DEPTH
        for k in range(min(_DEPTH, n_items)):
            in_flight[k % _DEPTH] = issue_gather(k)

        store_desc = [None] * _DEPTH
        for j in range(n_items):
            s = j % _DEPTH
            b, tc = j // n_sub, j % n_sub
            in_flight[s].wait()
            if store_desc[s] is not None:
                store_desc[s].wait()
                store_desc[s] = None

            def body(r, carry):
                for q in range(D // _LANES):
                    sl = pl.ds(q * _LANES, _LANES)
                    outs[s][r, sl] = gaths[s][r, sl] + pos_v[tc * _CHUNK + r, sl]
                return carry

            lax.fori_loop(0, _CHUNK, body, 0)
            if j + _DEPTH < n_items:
                in_flight[s] = issue_gather(j + _DEPTH)
            store_desc[s] = pltpu.async_copy(
                outs[s], out_hbm.at[pl.ds(b * T + t0 + tc * _CHUNK, _CHUNK)],
                ssems[s])

        for s in range(_DEPTH):
            if store_desc[s] is not None:
                store_desc[s].wait()

    return sc_kernel


def kernel(idx, wte, wpe):
    b, t = idx.shape
    v, d = wte.shape
    B = b * t
    idx_flat = idx.reshape(B).astype(jnp.int32)
    out = _build(B, t, d, b, _NUM_WORKERS)(idx_flat, wte, wpe)
    return out.reshape(b, t, d)
